# baseline (speedup 1.0000x reference)
import functools

import jax
import jax.numpy as jnp
from jax import lax
from jax.experimental import pallas as pl
from jax.experimental.pallas import tpu as pltpu
from jax.experimental.pallas import tpu_sc as plsc

N = 8192
ROWS = 4096
WPR = 2048
NC, NS, L = 2, 16, 16
NTILES = NC * NS
RPT = ROWS // NTILES
CH = 16
NCH = RPT // CH
HBINS = 8320
DUMP = HBINS - 1
SLICE = HBINS // NS
OUTB = N // NS
WEIGHT = 0.999

_mesh = plsc.VectorSubcoreMesh(core_axis_name="c", subcore_axis_name="s")


@functools.partial(
    pl.kernel,
    out_type=jax.ShapeDtypeStruct((NC * N,), jnp.float32),
    mesh=_mesh,
    compiler_params=pltpu.CompilerParams(use_tc_tiling_on_sc=False),
    scratch_types=[
        pltpu.VMEM((CH * WPR,), jnp.int32),
        pltpu.VMEM((CH * WPR,), jnp.int32),
        pltpu.VMEM((RPT, L), jnp.float32),
        pltpu.VMEM((RPT,), jnp.int32),
        pltpu.VMEM((2 * L,), jnp.int32),
        pltpu.VMEM((OUTB, L), jnp.float32),
        pltpu.VMEM((OUTB,), jnp.float32),
        pltpu.VMEM_SHARED((HBINS, L), jnp.float32),
        pltpu.SemaphoreType.DMA,
        pltpu.SemaphoreType.DMA,
    ],
)
def _sc_hist(m_hbm, zeros_hbm, ones_hbm, out_hbm,
             buf0, buf1, ones_v, idx_v, fold_v, wb_v, res_v, counts_sh,
             sem0, sem1):
    cid = lax.axis_index("c")
    sid = lax.axis_index("s")
    wid = cid * NS + sid
    row0 = wid * RPT

    pltpu.sync_copy(zeros_hbm.at[pl.ds(sid * SLICE, SLICE)],
                    counts_sh.at[pl.ds(sid * SLICE, SLICE)])
    pltpu.sync_copy(ones_hbm, ones_v)

    fold_v[pl.ds(L, L)] = jnp.zeros((L,), jnp.int32)

    bufs = (buf0, buf1)
    sems = (sem0, sem1)
    copies = [None, None]
    copies[0] = pltpu.async_copy(
        m_hbm.at[pl.ds(row0 * WPR, CH * WPR)], buf0, sem0)
    mask_lo = jnp.int32(0x00FF00FF)

    for c in range(NCH):
        cur = c % 2
        copies[cur].wait()
        if c + 1 < NCH:
            nxt = (c + 1) % 2
            copies[nxt] = pltpu.async_copy(
                m_hbm.at[pl.ds((row0 + (c + 1) * CH) * WPR, CH * WPR)],
                bufs[nxt], sems[nxt])
        buf = bufs[cur]
        lane = lax.broadcasted_iota(jnp.int32, (L,), 0)

        def row_body(rr, lv, buf=buf):
            def vec_body(j, acc):
                return acc + buf[pl.ds(rr * WPR + j * L, L)]
            acc = lax.fori_loop(0, WPR // L, vec_body,
                                jnp.zeros((L,), jnp.int32))
            s = (acc & mask_lo) + (lax.shift_right_logical(acc, 8) & mask_lo)
            cur = (s & 0xFFFF) + lax.shift_right_logical(s, 16)
            for sh in (8, 4, 2):
                fold_v[pl.ds(0, L)] = cur
                cur = cur + fold_v[pl.ds(sh, L)]
            total = cur[0] + cur[1]
            return jnp.where(lane == rr, total, lv)

        lv = lax.fori_loop(0, CH, row_body, jnp.zeros((L,), jnp.int32))
        idx_v[pl.ds(c * CH, CH)] = jnp.where(lv == 0, DUMP, lv - 1)

    plsc.subcore_barrier()
    pltpu.sync_copy(ones_v, counts_sh.at[idx_v], add=True)
    plsc.subcore_barrier()

    pltpu.sync_copy(counts_sh.at[pl.ds(sid * OUTB, OUTB)], wb_v)

    lane_f = lax.broadcasted_iota(jnp.int32, (L,), 0)

    def group_body(g, _):
        def pick_body(k, v):
            row = wb_v[g * L + k, pl.ds(0, L)]
            return jnp.where(lane_f == k, row[0], v)
        v = lax.fori_loop(0, L, pick_body, jnp.zeros((L,), jnp.float32))
        res_v[pl.ds(g * L, L)] = v
        return 0

    lax.fori_loop(0, OUTB // L, group_body, 0)
    pltpu.sync_copy(res_v, out_hbm.at[pl.ds(cid * N + sid * OUTB, OUTB)])


def _blend_kernel(parts_ref, p_ref, out_ref):
    c = parts_ref[0:1, :] + parts_ref[1:2, :]
    out_ref[...] = WEIGHT * p_ref[...] + ((1.0 - WEIGHT) / ROWS) * c


def kernel(mask, n_elements_prob):
    m32 = mask.view(jnp.int32).reshape(ROWS * WPR)
    zeros = jnp.zeros((HBINS, L), jnp.float32)
    ones = jnp.ones((RPT, L), jnp.float32)
    parts = _sc_hist(m32, zeros, ones).reshape(NC, N)
    out = pl.pallas_call(
        _blend_kernel,
        in_specs=[
            pl.BlockSpec((NC, N), lambda: (0, 0)),
            pl.BlockSpec((1, N), lambda: (0, 0)),
        ],
        out_specs=pl.BlockSpec((1, N), lambda: (0, 0)),
        out_shape=jax.ShapeDtypeStruct((1, N), jnp.float32),
    )(parts, n_elements_prob.reshape(1, N))
    return out.reshape(N)


# speedup vs baseline: 1.1290x; 1.1290x over previous
import functools

import jax
import jax.numpy as jnp
from jax import lax
from jax.experimental import pallas as pl
from jax.experimental.pallas import tpu as pltpu
from jax.experimental.pallas import tpu_sc as plsc

N = 8192
ROWS = 4096
WPR = 2048
NC, NS, L = 2, 16, 16
NTILES = NC * NS
RPT = ROWS // NTILES
CH = 8
NCH = RPT // CH
CW = 128
HBINS = 8320
DUMP = HBINS - 1
SLICE = HBINS // NS
OUTB = N // NS
WBC = 64
WEIGHT = 0.999

_mesh = plsc.VectorSubcoreMesh(core_axis_name="c", subcore_axis_name="s")


@functools.partial(
    pl.kernel,
    out_type=jax.ShapeDtypeStruct((NC * N,), jnp.float32),
    mesh=_mesh,
    scratch_types=[
        pltpu.VMEM((CH, WPR), jnp.int32),
        pltpu.VMEM((CH, WPR), jnp.int32),
        pltpu.VMEM((RPT, CW), jnp.float32),
        pltpu.VMEM((RPT,), jnp.int32),
        pltpu.VMEM((2 * L,), jnp.int32),
        pltpu.VMEM((WBC, CW), jnp.float32),
        pltpu.VMEM((OUTB,), jnp.float32),
        pltpu.VMEM_SHARED((HBINS, CW), jnp.float32),
        pltpu.SemaphoreType.DMA,
        pltpu.SemaphoreType.DMA,
    ],
)
def _sc_hist(m_hbm, zeros_hbm, ones_hbm, out_hbm,
             buf0, buf1, ones_v, idx_v, fold_v, wb_v, res_v, counts_sh,
             sem0, sem1):
    cid = lax.axis_index("c")
    sid = lax.axis_index("s")
    wid = cid * NS + sid
    row0 = wid * RPT

    pltpu.sync_copy(zeros_hbm.at[pl.ds(sid * SLICE, SLICE)],
                    counts_sh.at[pl.ds(sid * SLICE, SLICE)])
    pltpu.sync_copy(ones_hbm, ones_v)
    fold_v[pl.ds(L, L)] = jnp.zeros((L,), jnp.int32)

    bufs = (buf0, buf1)
    sems = (sem0, sem1)
    copies = [None, None]
    copies[0] = pltpu.async_copy(m_hbm.at[pl.ds(row0, CH)], buf0, sem0)
    mask_lo = jnp.int32(0x00FF00FF)
    lane = lax.broadcasted_iota(jnp.int32, (L,), 0)

    for c in range(NCH):
        cur_i = c % 2
        copies[cur_i].wait()
        if c + 1 < NCH:
            nxt = (c + 1) % 2
            copies[nxt] = pltpu.async_copy(
                m_hbm.at[pl.ds(row0 + (c + 1) * CH, CH)], bufs[nxt],
                sems[nxt])
        buf = bufs[cur_i]
        half = c % 2

        def row_body(rr, lv, buf=buf, half=half):
            def vec_body(j, accs):
                a0, a1, a2, a3 = accs
                base = j * (4 * L)
                return (a0 + buf[rr, pl.ds(base, L)],
                        a1 + buf[rr, pl.ds(base + L, L)],
                        a2 + buf[rr, pl.ds(base + 2 * L, L)],
                        a3 + buf[rr, pl.ds(base + 3 * L, L)])

            z = jnp.zeros((L,), jnp.int32)
            a0, a1, a2, a3 = lax.fori_loop(0, WPR // (4 * L), vec_body,
                                           (z, z, z, z), unroll=4)
            acc = (a0 + a1) + (a2 + a3)
            s = (acc & mask_lo) + (lax.shift_right_logical(acc, 8) & mask_lo)
            cur = (s & 0xFFFF) + lax.shift_right_logical(s, 16)
            for sh in (8, 4, 2):
                fold_v[pl.ds(0, L)] = cur
                cur = cur + fold_v[pl.ds(sh, L)]
            total = cur[0] + cur[1]
            return jnp.where(lane == rr + half * CH, total, lv)

        init = jnp.zeros((L,), jnp.int32) if half == 0 else lv_carry
        lv_carry = lax.fori_loop(0, CH, row_body, init)
        if half == 1:
            idx_v[pl.ds((c // 2) * L, L)] = jnp.where(
                lv_carry == 0, DUMP, lv_carry - 1)

    plsc.subcore_barrier()
    pltpu.sync_copy(ones_v, counts_sh.at[idx_v], add=True)
    plsc.subcore_barrier()

    for q in range(OUTB // WBC):
        pltpu.sync_copy(
            counts_sh.at[pl.ds(sid * OUTB + q * WBC, WBC)], wb_v)

        def group_body(g, _, q=q):
            def pick_body(k, v):
                row = wb_v[g * L + k, pl.ds(0, L)]
                return jnp.where(lane == k, row[0], v)
            v = lax.fori_loop(0, L, pick_body, jnp.zeros((L,), jnp.float32))
            res_v[pl.ds(q * WBC + g * L, L)] = v
            return 0

        lax.fori_loop(0, WBC // L, group_body, 0)

    pltpu.sync_copy(res_v, out_hbm.at[pl.ds(cid * N + sid * OUTB, OUTB)])


def _blend_kernel(parts_ref, p_ref, out_ref):
    c = parts_ref[pl.ds(0, N)] + parts_ref[pl.ds(N, N)]
    out_ref[...] = WEIGHT * p_ref[...] + ((1.0 - WEIGHT) / ROWS) * c


def kernel(mask, n_elements_prob):
    m32 = mask.view(jnp.int32)
    zeros = jnp.zeros((HBINS, CW), jnp.float32)
    ones = jnp.ones((RPT, CW), jnp.float32)
    parts = _sc_hist(m32, zeros, ones)
    out = pl.pallas_call(
        _blend_kernel,
        in_specs=[
            pl.BlockSpec((NC * N,), lambda: (0,)),
            pl.BlockSpec((N,), lambda: (0,)),
        ],
        out_specs=pl.BlockSpec((N,), lambda: (0,)),
        out_shape=jax.ShapeDtypeStruct((N,), jnp.float32),
    )(parts, n_elements_prob)
    return out


# speedup vs baseline: 1.1385x; 1.0085x over previous
import functools

import jax
import jax.numpy as jnp
from jax import lax
from jax.experimental import pallas as pl
from jax.experimental.pallas import tpu as pltpu
from jax.experimental.pallas import tpu_sc as plsc

N = 8192
ROWS = 4096
WPR = 2048
NC, NS, L = 2, 16, 16
NTILES = NC * NS
RPT = ROWS // NTILES
CH = 8
NCH = RPT // CH
CW = 128
HBINS = 8320
DUMP = HBINS - 1
SLICE = HBINS // NS
OUTB = N // NS
WBC = 64
WEIGHT = 0.999

_mesh = plsc.VectorSubcoreMesh(core_axis_name="c", subcore_axis_name="s")


@functools.partial(
    pl.kernel,
    out_type=jax.ShapeDtypeStruct((NC * N,), jnp.float32),
    mesh=_mesh,
    compiler_params=pltpu.CompilerParams(use_tc_tiling_on_sc=True),
    scratch_types=[
        pltpu.VMEM((CH, WPR), jnp.int32),
        pltpu.VMEM((CH, WPR), jnp.int32),
        pltpu.VMEM((RPT, CW), jnp.float32),
        pltpu.VMEM((RPT,), jnp.int32),
        pltpu.VMEM((2 * L,), jnp.int32),
        pltpu.VMEM((WBC, CW), jnp.float32),
        pltpu.VMEM((OUTB,), jnp.float32),
        pltpu.VMEM_SHARED((HBINS, CW), jnp.float32),
        pltpu.SemaphoreType.DMA,
        pltpu.SemaphoreType.DMA,
    ],
)
def _sc_hist(m_hbm, zeros_hbm, ones_hbm, out_hbm,
             buf0, buf1, ones_v, idx_v, fold_v, wb_v, res_v, counts_sh,
             sem0, sem1):
    cid = lax.axis_index("c")
    sid = lax.axis_index("s")
    wid = cid * NS + sid
    row0 = wid * RPT

    pltpu.sync_copy(zeros_hbm.at[pl.ds(sid * SLICE, SLICE)],
                    counts_sh.at[pl.ds(sid * SLICE, SLICE)])
    pltpu.sync_copy(ones_hbm, ones_v)
    fold_v[pl.ds(L, L)] = jnp.zeros((L,), jnp.int32)

    bufs = (buf0, buf1)
    sems = (sem0, sem1)
    copies = [None, None]
    copies[0] = pltpu.async_copy(m_hbm.at[pl.ds(row0, CH)], buf0, sem0)
    mask_lo = jnp.int32(0x00FF00FF)
    lane = lax.broadcasted_iota(jnp.int32, (L,), 0)

    for c in range(NCH):
        cur_i = c % 2
        copies[cur_i].wait()
        if c + 1 < NCH:
            nxt = (c + 1) % 2
            copies[nxt] = pltpu.async_copy(
                m_hbm.at[pl.ds(row0 + (c + 1) * CH, CH)], bufs[nxt],
                sems[nxt])
        buf = bufs[cur_i]
        half = c % 2

        def row_body(rr, lv, buf=buf, half=half):
            def vec_body(j, accs):
                a0, a1, a2, a3 = accs
                base = j * (4 * L)
                return (a0 + buf[rr, pl.ds(base, L)],
                        a1 + buf[rr, pl.ds(base + L, L)],
                        a2 + buf[rr, pl.ds(base + 2 * L, L)],
                        a3 + buf[rr, pl.ds(base + 3 * L, L)])

            z = jnp.zeros((L,), jnp.int32)
            a0, a1, a2, a3 = lax.fori_loop(0, WPR // (4 * L), vec_body,
                                           (z, z, z, z), unroll=4)
            acc = (a0 + a1) + (a2 + a3)
            s = (acc & mask_lo) + (lax.shift_right_logical(acc, 8) & mask_lo)
            cur = (s & 0xFFFF) + lax.shift_right_logical(s, 16)
            for sh in (8, 4, 2):
                fold_v[pl.ds(0, L)] = cur
                cur = cur + fold_v[pl.ds(sh, L)]
            total = cur[0] + cur[1]
            return jnp.where(lane == rr + half * CH, total, lv)

        init = jnp.zeros((L,), jnp.int32) if half == 0 else lv_carry
        lv_carry = lax.fori_loop(0, CH, row_body, init)
        if half == 1:
            idx_v[pl.ds((c // 2) * L, L)] = jnp.where(
                lv_carry == 0, DUMP, lv_carry - 1)

    plsc.subcore_barrier()
    pltpu.sync_copy(ones_v, counts_sh.at[idx_v], add=True)
    plsc.subcore_barrier()

    for q in range(OUTB // WBC):
        pltpu.sync_copy(
            counts_sh.at[pl.ds(sid * OUTB + q * WBC, WBC)], wb_v)

        def group_body(g, _, q=q):
            def pick_body(k, v):
                row = wb_v[g * L + k, pl.ds(0, L)]
                return jnp.where(lane == k, row[0], v)
            v = lax.fori_loop(0, L, pick_body, jnp.zeros((L,), jnp.float32))
            res_v[pl.ds(q * WBC + g * L, L)] = v
            return 0

        lax.fori_loop(0, WBC // L, group_body, 0)

    pltpu.sync_copy(res_v, out_hbm.at[pl.ds(cid * N + sid * OUTB, OUTB)])


def _blend_kernel(parts_ref, p_ref, out_ref):
    c = parts_ref[pl.ds(0, N)] + parts_ref[pl.ds(N, N)]
    out_ref[...] = WEIGHT * p_ref[...] + ((1.0 - WEIGHT) / ROWS) * c


def kernel(mask, n_elements_prob):
    m32 = mask.view(jnp.int32)
    zeros = jnp.zeros((HBINS, CW), jnp.float32)
    ones = jnp.ones((RPT, CW), jnp.float32)
    parts = _sc_hist(m32, zeros, ones)
    out = pl.pallas_call(
        _blend_kernel,
        in_specs=[
            pl.BlockSpec((NC * N,), lambda: (0,)),
            pl.BlockSpec((N,), lambda: (0,)),
        ],
        out_specs=pl.BlockSpec((N,), lambda: (0,)),
        out_shape=jax.ShapeDtypeStruct((N,), jnp.float32),
    )(parts, n_elements_prob)
    return out


# speedup vs baseline: 11.6923x; 10.2699x over previous
import jax
import jax.numpy as jnp
from jax.experimental import pallas as pl

N = 8192
ROWS = 4096
BLK = 1024
HI = 128
LO = 64
WEIGHT = 0.999


def _hist_kernel(ml_ref, mr_ref, p_ref, out_ref):
    i = pl.program_id(0)
    ones = jnp.ones((N // 2, 1), dtype=jnp.int8)
    lens_l = jax.lax.dot_general(
        ml_ref[...], ones, (((1,), (0,)), ((), ())),
        preferred_element_type=jnp.int32)
    lens_r = jax.lax.dot_general(
        mr_ref[...], ones, (((1,), (0,)), ((), ())),
        preferred_element_type=jnp.int32)
    t = lens_l + lens_r - 1
    hi = t >> 6
    lo = t & (LO - 1)
    hiota = jax.lax.broadcasted_iota(jnp.int32, (1, HI), 1)
    loiota = jax.lax.broadcasted_iota(jnp.int32, (1, LO), 1)
    u = (hi == hiota).astype(jnp.bfloat16)
    v = (lo == loiota).astype(jnp.bfloat16)
    part = jax.lax.dot_general(
        u, v, (((0,), (0,)), ((), ())),
        preferred_element_type=jnp.float32)

    @pl.when(i == 0)
    def _init():
        out_ref[...] = jnp.zeros_like(out_ref)

    out_ref[...] += part

    @pl.when(i == pl.num_programs(0) - 1)
    def _finish():
        out_ref[...] = WEIGHT * p_ref[...] + ((1.0 - WEIGHT) / ROWS) * out_ref[...]


def kernel(mask, n_elements_prob):
    m8 = mask.view(jnp.int8)
    p2 = n_elements_prob.reshape(HI, LO)
    out = pl.pallas_call(
        _hist_kernel,
        grid=(ROWS // BLK,),
        in_specs=[
            pl.BlockSpec((BLK, N // 2), lambda i: (i, 0)),
            pl.BlockSpec((BLK, N // 2), lambda i: (i, 1)),
            pl.BlockSpec((HI, LO), lambda i: (0, 0)),
        ],
        out_specs=pl.BlockSpec((HI, LO), lambda i: (0, 0)),
        out_shape=jax.ShapeDtypeStruct((HI, LO), jnp.float32),
    )(m8, m8, p2)
    return out.reshape(N)


# speedup vs baseline: 11.6975x; 1.0004x over previous
import jax
import jax.numpy as jnp
from jax.experimental import pallas as pl
from jax.experimental.pallas import tpu as pltpu

N = 8192
ROWS = 4096
BLK = 1024
HI = 128
LO = 64
WEIGHT = 0.999


def _hist_kernel(ml_ref, mr_ref, p_ref, out_ref):
    i = pl.program_id(0)
    ones = jnp.ones((N // 2, 1), dtype=jnp.int8)
    lens_l = jax.lax.dot_general(
        ml_ref[...], ones, (((1,), (0,)), ((), ())),
        preferred_element_type=jnp.int32)
    lens_r = jax.lax.dot_general(
        mr_ref[...], ones, (((1,), (0,)), ((), ())),
        preferred_element_type=jnp.int32)
    t = lens_l + lens_r - 1
    hi = t >> 6
    lo = t & (LO - 1)
    hiota = jax.lax.broadcasted_iota(jnp.int32, (1, HI), 1)
    loiota = jax.lax.broadcasted_iota(jnp.int32, (1, LO), 1)
    u = (hi == hiota).astype(jnp.bfloat16)
    v = (lo == loiota).astype(jnp.bfloat16)
    part = jax.lax.dot_general(
        u, v, (((0,), (0,)), ((), ())),
        preferred_element_type=jnp.float32)

    @pl.when(i == 0)
    def _init():
        out_ref[...] = jnp.zeros_like(out_ref)

    out_ref[...] += part

    @pl.when(i == pl.num_programs(0) - 1)
    def _finish():
        out_ref[...] = WEIGHT * p_ref[...] + ((1.0 - WEIGHT) / ROWS) * out_ref[...]


def kernel(mask, n_elements_prob):
    m8 = mask.view(jnp.int8)
    p2 = n_elements_prob.reshape(HI, LO)
    out = pl.pallas_call(
        _hist_kernel,
        grid=(ROWS // BLK,),
        in_specs=[
            pl.BlockSpec((BLK, N // 2), lambda i: (i, 0)),
            pl.BlockSpec((BLK, N // 2), lambda i: (i, 1)),
            pl.BlockSpec((HI, LO), lambda i: (0, 0)),
        ],
        out_specs=pl.BlockSpec((HI, LO), lambda i: (0, 0)),
        compiler_params=pltpu.CompilerParams(
            dimension_semantics=("arbitrary",)),
        out_shape=jax.ShapeDtypeStruct((HI, LO), jnp.float32),
    )(m8, m8, p2)
    return out.reshape(N)
